# baseline (speedup 1.0000x reference)
import functools

import jax
import jax.numpy as jnp
from jax import lax
from jax.experimental import pallas as pl
from jax.experimental.pallas import tpu as pltpu
from jax.experimental.pallas import tpu_sc as plsc

N_ROWS = 8388608
NUM_CORES = 2
NUM_SUBCORES = 16
NUM_WORKERS = NUM_CORES * NUM_SUBCORES
ROWS_PER_WORKER = N_ROWS // NUM_WORKERS
CHUNK = 8192
NUM_CHUNKS = ROWS_PER_WORKER // CHUNK
GROUPS = CHUNK // 128
LANES = 16


def _compute_chunk(xin, yout, s0, s1):
    @plsc.parallel_loop(0, GROUPS, unroll=4)
    def group_body(g):
        bi = g * 256
        bo = g * 128
        for j in range(128 // LANES):
            a = xin[pl.ds(bi + j * LANES, LANES)]
            b = xin[pl.ds(bi + 128 + j * LANES, LANES)]
            d = b - a
            z = jnp.where(d > s0, jnp.where(d > s1, 4, 2), 1)
            yout[pl.ds(bo + j * LANES, LANES)] = z


def _sc_body(x_hbm, s0_hbm, s1_hbm, out_hbm, xins, youts, s0_ref, s1_ref,
             in_sems, out_sems):
    cid = lax.axis_index("c")
    sid = lax.axis_index("s")
    wid = sid * NUM_CORES + cid
    base_row = wid * ROWS_PER_WORKER

    def in_copy(k, buf):
        row0 = base_row + k * CHUNK
        return pltpu.make_async_copy(
            x_hbm.at[pl.ds(row0 * 2, CHUNK * 2)], xins[buf], in_sems[buf]
        )

    def out_copy(k, buf):
        row0 = base_row + k * CHUNK
        return pltpu.make_async_copy(
            youts[buf],
            out_hbm.at[pl.ds(row0, CHUNK)],
            out_sems[buf],
        )

    in_copy(0, 0).start()
    pltpu.sync_copy(s0_hbm, s0_ref)
    pltpu.sync_copy(s1_hbm, s1_ref)
    s0 = s0_ref[:]
    s1 = s1_ref[:]

    def pair_body(m, carry):
        for buf in (0, 1):
            k = m * 2 + buf
            in_copy(k, buf).wait()

            @pl.when(k + 1 < NUM_CHUNKS)
            def _():
                in_copy(k + 1, 1 - buf).start()

            @pl.when(m > 0)
            def _():
                out_copy(k - 2, buf).wait()

            _compute_chunk(xins[buf], youts[buf], s0, s1)
            out_copy(k, buf).start()
        return carry

    lax.fori_loop(0, NUM_CHUNKS // 2, pair_body, 0)
    out_copy(NUM_CHUNKS - 2, 0).wait()
    out_copy(NUM_CHUNKS - 1, 1).wait()


@functools.partial(
    pl.kernel,
    out_type=jax.ShapeDtypeStruct((N_ROWS,), jnp.int32),
    mesh=plsc.VectorSubcoreMesh(core_axis_name="c", subcore_axis_name="s"),
    compiler_params=pltpu.CompilerParams(needs_layout_passes=False),
    scratch_types=[
        [pltpu.VMEM((CHUNK * 2,), jnp.float32) for _ in range(2)],
        [pltpu.VMEM((CHUNK,), jnp.int32) for _ in range(2)],
        pltpu.VMEM((LANES,), jnp.float32),
        pltpu.VMEM((LANES,), jnp.float32),
        [pltpu.SemaphoreType.DMA for _ in range(2)],
        [pltpu.SemaphoreType.DMA for _ in range(2)],
    ],
)
def _sc_kernel(x_hbm, s0_hbm, s1_hbm, out_hbm, xins, youts, s0_ref, s1_ref,
               in_sems, out_sems):
    _sc_body(x_hbm, s0_hbm, s1_hbm, out_hbm, xins, youts, s0_ref, s1_ref,
             in_sems, out_sems)


def kernel(coordinates, splits):
    x = coordinates.reshape(N_ROWS // 128, 128, 2).transpose(0, 2, 1).reshape(-1)
    s0 = jnp.broadcast_to(splits[0], (LANES,))
    s1 = jnp.broadcast_to(splits[1], (LANES,))
    y = _sc_kernel(x, s0, s1)
    return (y[:, None] >> jnp.arange(3, dtype=jnp.int32)[None, :]) & 1


# speedup vs baseline: 1.0950x; 1.0950x over previous
import functools

import jax
import jax.numpy as jnp
from jax import lax
from jax.experimental import pallas as pl
from jax.experimental.pallas import tpu as pltpu
from jax.experimental.pallas import tpu_sc as plsc

N_ROWS = 8388608
NUM_CORES = 2
NUM_SUBCORES = 16
NUM_WORKERS = NUM_CORES * NUM_SUBCORES
ROWS_PER_WORKER = N_ROWS // NUM_WORKERS
CHUNK = 16384
NUM_CHUNKS = ROWS_PER_WORKER // CHUNK
GROUPS = CHUNK // 128
LANES = 16


def _compute_chunk(xin, yout, s0, s1):
    @plsc.parallel_loop(0, GROUPS, unroll=4)
    def group_body(g):
        bi = g * 256
        bo = g * 128
        for j in range(128 // LANES):
            a = xin[pl.ds(bi + j * LANES, LANES)]
            b = xin[pl.ds(bi + 128 + j * LANES, LANES)]
            d = b - a
            z = jnp.where(d > s0, jnp.where(d > s1, 4, 2), 1)
            yout[pl.ds(bo + j * LANES, LANES)] = z


def _sc_body(x_hbm, s0_hbm, s1_hbm, out_hbm, xins, youts, s0_ref, s1_ref,
             in_sems, out_sems):
    cid = lax.axis_index("c")
    sid = lax.axis_index("s")
    wid = sid * NUM_CORES + cid
    base_row = wid * ROWS_PER_WORKER

    def in_copy(k, buf):
        row0 = base_row + k * CHUNK
        return pltpu.make_async_copy(
            x_hbm.at[pl.ds(row0 * 2, CHUNK * 2)], xins[buf], in_sems[buf]
        )

    def out_copy(k, buf):
        row0 = base_row + k * CHUNK
        return pltpu.make_async_copy(
            youts[buf],
            out_hbm.at[pl.ds(row0, CHUNK)],
            out_sems[buf],
        )

    in_copy(0, 0).start()
    pltpu.sync_copy(s0_hbm, s0_ref)
    pltpu.sync_copy(s1_hbm, s1_ref)
    s0 = s0_ref[:]
    s1 = s1_ref[:]

    def pair_body(m, carry):
        for buf in (0, 1):
            k = m * 2 + buf
            in_copy(k, buf).wait()

            @pl.when(k + 1 < NUM_CHUNKS)
            def _():
                in_copy(k + 1, 1 - buf).start()

            @pl.when(m > 0)
            def _():
                out_copy(k - 2, buf).wait()

            _compute_chunk(xins[buf], youts[buf], s0, s1)
            out_copy(k, buf).start()
        return carry

    lax.fori_loop(0, NUM_CHUNKS // 2, pair_body, 0)
    out_copy(NUM_CHUNKS - 2, 0).wait()
    out_copy(NUM_CHUNKS - 1, 1).wait()


@functools.partial(
    pl.kernel,
    out_type=jax.ShapeDtypeStruct((N_ROWS,), jnp.int32),
    mesh=plsc.VectorSubcoreMesh(core_axis_name="c", subcore_axis_name="s"),
    compiler_params=pltpu.CompilerParams(needs_layout_passes=False),
    scratch_types=[
        [pltpu.VMEM((CHUNK * 2,), jnp.float32) for _ in range(2)],
        [pltpu.VMEM((CHUNK,), jnp.int32) for _ in range(2)],
        pltpu.VMEM((LANES,), jnp.float32),
        pltpu.VMEM((LANES,), jnp.float32),
        [pltpu.SemaphoreType.DMA for _ in range(2)],
        [pltpu.SemaphoreType.DMA for _ in range(2)],
    ],
)
def _sc_kernel(x_hbm, s0_hbm, s1_hbm, out_hbm, xins, youts, s0_ref, s1_ref,
               in_sems, out_sems):
    _sc_body(x_hbm, s0_hbm, s1_hbm, out_hbm, xins, youts, s0_ref, s1_ref,
             in_sems, out_sems)


def kernel(coordinates, splits):
    x = coordinates.reshape(N_ROWS // 128, 128, 2).transpose(0, 2, 1).reshape(-1)
    s0 = jnp.broadcast_to(splits[0], (LANES,))
    s1 = jnp.broadcast_to(splits[1], (LANES,))
    y = _sc_kernel(x, s0, s1)
    return (y[:, None] >> jnp.arange(3, dtype=jnp.int32)[None, :]) & 1


# speedup vs baseline: 1.1156x; 1.0188x over previous
import functools

import jax
import jax.numpy as jnp
from jax import lax
from jax.experimental import pallas as pl
from jax.experimental.pallas import tpu as pltpu
from jax.experimental.pallas import tpu_sc as plsc

N_ROWS = 8388608
NUM_CORES = 2
NUM_SUBCORES = 16
NUM_WORKERS = NUM_CORES * NUM_SUBCORES
ROWS_PER_WORKER = N_ROWS // NUM_WORKERS
CHUNK = 16384
NUM_CHUNKS = ROWS_PER_WORKER // CHUNK
GROUPS = CHUNK // 128
LANES = 16


def _compute_chunk(xin, yout, s0, s1):
    @plsc.parallel_loop(0, GROUPS, unroll=8)
    def group_body(g):
        bi = g * 256
        bo = g * 128
        for j in range(128 // LANES):
            a = xin[pl.ds(bi + j * LANES, LANES)]
            b = xin[pl.ds(bi + 128 + j * LANES, LANES)]
            d = b - a
            z = jnp.where(d > s0, jnp.where(d > s1, 4, 2), 1)
            yout[pl.ds(bo + j * LANES, LANES)] = z


def _sc_body(x_hbm, s0_hbm, s1_hbm, out_hbm, xins, youts, s0_ref, s1_ref,
             in_sems, out_sems):
    cid = lax.axis_index("c")
    sid = lax.axis_index("s")
    wid = sid * NUM_CORES + cid
    base_row = wid * ROWS_PER_WORKER

    def in_copy(k, buf):
        row0 = base_row + k * CHUNK
        return pltpu.make_async_copy(
            x_hbm.at[pl.ds(row0 * 2, CHUNK * 2)], xins[buf], in_sems[buf]
        )

    def out_copy(k, buf):
        row0 = base_row + k * CHUNK
        return pltpu.make_async_copy(
            youts[buf],
            out_hbm.at[pl.ds(row0, CHUNK)],
            out_sems[buf],
        )

    in_copy(0, 0).start()
    pltpu.sync_copy(s0_hbm, s0_ref)
    pltpu.sync_copy(s1_hbm, s1_ref)
    s0 = s0_ref[:]
    s1 = s1_ref[:]

    def pair_body(m, carry):
        for buf in (0, 1):
            k = m * 2 + buf

            @pl.when(k + 1 < NUM_CHUNKS)
            def _():
                in_copy(k + 1, 1 - buf).start()

            in_copy(k, buf).wait()

            @pl.when(m > 0)
            def _():
                out_copy(k - 2, buf).wait()

            _compute_chunk(xins[buf], youts[buf], s0, s1)
            out_copy(k, buf).start()
        return carry

    lax.fori_loop(0, NUM_CHUNKS // 2, pair_body, 0)
    out_copy(NUM_CHUNKS - 2, 0).wait()
    out_copy(NUM_CHUNKS - 1, 1).wait()


@functools.partial(
    pl.kernel,
    out_type=jax.ShapeDtypeStruct((N_ROWS,), jnp.int32),
    mesh=plsc.VectorSubcoreMesh(core_axis_name="c", subcore_axis_name="s"),
    compiler_params=pltpu.CompilerParams(needs_layout_passes=False),
    scratch_types=[
        [pltpu.VMEM((CHUNK * 2,), jnp.float32) for _ in range(2)],
        [pltpu.VMEM((CHUNK,), jnp.int32) for _ in range(2)],
        pltpu.VMEM((LANES,), jnp.float32),
        pltpu.VMEM((LANES,), jnp.float32),
        [pltpu.SemaphoreType.DMA for _ in range(2)],
        [pltpu.SemaphoreType.DMA for _ in range(2)],
    ],
)
def _sc_kernel(x_hbm, s0_hbm, s1_hbm, out_hbm, xins, youts, s0_ref, s1_ref,
               in_sems, out_sems):
    _sc_body(x_hbm, s0_hbm, s1_hbm, out_hbm, xins, youts, s0_ref, s1_ref,
             in_sems, out_sems)


def kernel(coordinates, splits):
    x = coordinates.reshape(N_ROWS // 128, 128, 2).transpose(0, 2, 1).reshape(-1)
    s0 = jnp.broadcast_to(splits[0], (LANES,))
    s1 = jnp.broadcast_to(splits[1], (LANES,))
    y = _sc_kernel(x, s0, s1)
    return (y[:, None] >> jnp.arange(3, dtype=jnp.int32)[None, :]) & 1
